# baseline (speedup 1.0000x reference)
import jax
import jax.numpy as jnp
from jax import lax
from jax.experimental import pallas as pl
from jax.experimental.pallas import tpu as pltpu

V = 100000
D = 64
C = 200
H = 128

BW = 8192
NB = (V + BW - 1) // BW
V_FLOOR = (NB - 1) * BW
V_TAIL = V - V_FLOOR


def _one_hot_row(idx_scalar, blk):
    lane = lax.broadcasted_iota(jnp.int32, (1, 128), 1)
    onehot = (lane == (idx_scalar % 128)).astype(jnp.float32)
    return lax.dot_general(
        onehot, blk,
        dimension_numbers=(((1,), (1,)), ((), ())),
        preferred_element_type=jnp.float32,
    )


def _fused_body(idx_ref, *refs):
    blk_refs = refs[:C]
    w1_ref, b1_ref, w2t_ref, b2_ref, out_ref, h_ref = refs[C:]
    j = pl.program_id(0)

    @pl.when(j == 0)
    def _():
        rows = [_one_hot_row(idx_ref[k], blk_refs[k][...]) for k in range(C)]
        x = jnp.concatenate(rows, axis=1)
        h = jnp.dot(x, w1_ref[...], preferred_element_type=jnp.float32)
        h_ref[...] = jnp.maximum(h + b1_ref[...], 0.0)

    logits = (
        lax.dot_general(
            h_ref[...], w2t_ref[...],
            dimension_numbers=(((1,), (1,)), ((), ())),
            preferred_element_type=jnp.float32,
        )
        + b2_ref[...]
    )

    @pl.when(j < NB - 1)
    def _():
        out_ref[:, pl.ds(pl.multiple_of(j * BW, BW), BW)] = logits

    @pl.when(j == NB - 1)
    def _():
        out_ref[:, V_FLOOR:V] = logits[:, :V_TAIL]
        x = out_ref[...]
        m = jnp.max(x)
        s = jnp.sum(jnp.exp(x - m))
        out_ref[...] = x - (m + jnp.log(s))


def _gather_spec(k):
    return pl.BlockSpec((D, 128), lambda j, idx: (0, idx[k] // 128))


_fused_call = pl.pallas_call(
    _fused_body,
    grid_spec=pltpu.PrefetchScalarGridSpec(
        num_scalar_prefetch=1,
        grid=(NB,),
        in_specs=(
            [_gather_spec(k) for k in range(C)]
            + [
                pl.BlockSpec((C * D, H), lambda j, idx: (0, 0)),
                pl.BlockSpec((1, H), lambda j, idx: (0, 0)),
                pl.BlockSpec((BW, H), lambda j, idx: (j, 0)),
                pl.BlockSpec((1, BW), lambda j, idx: (0, j)),
            ]
        ),
        out_specs=pl.BlockSpec((1, V), lambda j, idx: (0, 0)),
        scratch_shapes=[pltpu.VMEM((1, H), jnp.float32)],
    ),
    out_shape=jax.ShapeDtypeStruct((1, V), jnp.float32),
    compiler_params=pltpu.CompilerParams(
        dimension_semantics=("arbitrary",),
    ),
)


def kernel(inputs, emb, W1, b1, W2, b2):
    embT = emb.T
    idx = inputs.astype(jnp.int32)
    return _fused_call(idx, *([embT] * C), W1, b1.reshape(1, H), W2.T,
                       b2.reshape(1, V))


# speedup vs baseline: 1.8281x; 1.8281x over previous
import jax
import jax.numpy as jnp
from jax import lax
from jax.experimental import pallas as pl
from jax.experimental.pallas import tpu as pltpu

V = 100000
D = 64
C = 200
H = 128

BW = 8192
NB = (V + BW - 1) // BW
V_FLOOR = (NB - 1) * BW
V_TAIL = V - V_FLOOR


def _one_hot_row(idx_scalar, blk):
    lane = lax.broadcasted_iota(jnp.int32, (1, 128), 1)
    onehot = (lane == (idx_scalar % 128)).astype(jnp.float32)
    return lax.dot_general(
        onehot, blk,
        dimension_numbers=(((1,), (1,)), ((), ())),
        preferred_element_type=jnp.float32,
    )


_GPER = 200


def _gather_body(idx_ref, *refs):
    t = pl.program_id(0)
    blk_refs, out_ref = refs[:_GPER], refs[_GPER]
    rows = [
        _one_hot_row(idx_ref[_GPER * t + k], blk_refs[k][...])
        for k in range(_GPER)
    ]
    out_ref[...] = jnp.concatenate(rows, axis=1)


def _gather_spec(k):
    return pl.BlockSpec((D, 128), lambda t, idx: (0, idx[_GPER * t + k] // 128))


_gather_call = pl.pallas_call(
    _gather_body,
    grid_spec=pltpu.PrefetchScalarGridSpec(
        num_scalar_prefetch=1,
        grid=(C // _GPER,),
        in_specs=[_gather_spec(k) for k in range(_GPER)],
        out_specs=pl.BlockSpec((1, _GPER * D), lambda t, idx: (0, t)),
    ),
    out_shape=jax.ShapeDtypeStruct((1, C * D), jnp.float32),
    compiler_params=pltpu.CompilerParams(
        dimension_semantics=("parallel",),
    ),
)


def _mlp_body(emb_ref, w1_ref, b1_ref, w2t_ref, b2_ref, out_ref, h_ref):
    j = pl.program_id(0)

    @pl.when(j == 0)
    def _():
        h = jnp.dot(emb_ref[...], w1_ref[...], preferred_element_type=jnp.float32)
        h_ref[...] = jnp.maximum(h + b1_ref[...], 0.0)

    logits = (
        lax.dot_general(
            h_ref[...], w2t_ref[...],
            dimension_numbers=(((1,), (1,)), ((), ())),
            preferred_element_type=jnp.float32,
        )
        + b2_ref[...]
    )

    @pl.when(j < NB - 1)
    def _():
        out_ref[:, pl.ds(pl.multiple_of(j * BW, BW), BW)] = logits

    @pl.when(j == NB - 1)
    def _():
        out_ref[:, V_FLOOR:V] = logits[:, :V_TAIL]
        x = out_ref[...]
        m = jnp.max(x)
        s = jnp.sum(jnp.exp(x - m))
        out_ref[...] = x - (m + jnp.log(s))


_mlp_call = pl.pallas_call(
    _mlp_body,
    grid=(NB,),
    in_specs=[
        pl.BlockSpec((1, C * D), lambda j: (0, 0)),
        pl.BlockSpec((C * D, H), lambda j: (0, 0)),
        pl.BlockSpec((1, H), lambda j: (0, 0)),
        pl.BlockSpec((BW, H), lambda j: (j, 0)),
        pl.BlockSpec((1, BW), lambda j: (0, j)),
    ],
    out_specs=pl.BlockSpec((1, V), lambda j: (0, 0)),
    out_shape=jax.ShapeDtypeStruct((1, V), jnp.float32),
    scratch_shapes=[pltpu.VMEM((1, H), jnp.float32)],
    compiler_params=pltpu.CompilerParams(
        dimension_semantics=("arbitrary",),
    ),
)


def kernel(inputs, emb, W1, b1, W2, b2):
    embT = emb.T
    idx = inputs.astype(jnp.int32)
    embeds = _gather_call(idx, *([embT] * _GPER))
    return _mlp_call(embeds, W1, b1.reshape(1, H), W2.T,
                     b2.reshape(1, V))


# speedup vs baseline: 1.9478x; 1.0655x over previous
import jax
import jax.numpy as jnp
from jax import lax
from jax.experimental import pallas as pl
from jax.experimental.pallas import tpu as pltpu

V = 100000
D = 64
C = 200
H = 128

BW = 16384
NB = (V + BW - 1) // BW
V_FLOOR = (NB - 1) * BW
V_TAIL = V - V_FLOOR


def _one_hot_row(idx_scalar, blk):
    lane = lax.broadcasted_iota(jnp.int32, (1, 128), 1)
    onehot = (lane == (idx_scalar % 128)).astype(jnp.float32)
    return lax.dot_general(
        onehot, blk,
        dimension_numbers=(((1,), (1,)), ((), ())),
        preferred_element_type=jnp.float32,
    )


_GPER = 200


def _gather_body(idx_ref, *refs):
    t = pl.program_id(0)
    blk_refs, out_ref = refs[:_GPER], refs[_GPER]
    rows = [
        _one_hot_row(idx_ref[_GPER * t + k], blk_refs[k][...])
        for k in range(_GPER)
    ]
    out_ref[...] = jnp.concatenate(rows, axis=1)


def _gather_spec(k):
    return pl.BlockSpec((D, 128), lambda t, idx: (0, idx[_GPER * t + k] // 128))


_gather_call = pl.pallas_call(
    _gather_body,
    grid_spec=pltpu.PrefetchScalarGridSpec(
        num_scalar_prefetch=1,
        grid=(C // _GPER,),
        in_specs=[_gather_spec(k) for k in range(_GPER)],
        out_specs=pl.BlockSpec((1, _GPER * D), lambda t, idx: (0, t)),
    ),
    out_shape=jax.ShapeDtypeStruct((1, C * D), jnp.float32),
    compiler_params=pltpu.CompilerParams(
        dimension_semantics=("parallel",),
    ),
)


def _mlp_body(emb_ref, w1_ref, b1_ref, w2t_ref, b2_ref, out_ref, h_ref):
    j = pl.program_id(0)

    @pl.when(j == 0)
    def _():
        h = jnp.dot(emb_ref[...], w1_ref[...], preferred_element_type=jnp.float32)
        h_ref[...] = jnp.maximum(h + b1_ref[...], 0.0)

    logits = (
        lax.dot_general(
            h_ref[...], w2t_ref[...],
            dimension_numbers=(((1,), (1,)), ((), ())),
            preferred_element_type=jnp.float32,
        )
        + b2_ref[...]
    )

    @pl.when(j < NB - 1)
    def _():
        out_ref[:, pl.ds(pl.multiple_of(j * BW, BW), BW)] = logits

    @pl.when(j == NB - 1)
    def _():
        out_ref[:, V_FLOOR:V] = logits[:, :V_TAIL]
        x = out_ref[...]
        m = jnp.max(x)
        s = jnp.sum(jnp.exp(x - m))
        out_ref[...] = x - (m + jnp.log(s))


_mlp_call = pl.pallas_call(
    _mlp_body,
    grid=(NB,),
    in_specs=[
        pl.BlockSpec((1, C * D), lambda j: (0, 0)),
        pl.BlockSpec((C * D, H), lambda j: (0, 0)),
        pl.BlockSpec((1, H), lambda j: (0, 0)),
        pl.BlockSpec((BW, H), lambda j: (j, 0)),
        pl.BlockSpec((1, BW), lambda j: (0, j)),
    ],
    out_specs=pl.BlockSpec((1, V), lambda j: (0, 0)),
    out_shape=jax.ShapeDtypeStruct((1, V), jnp.float32),
    scratch_shapes=[pltpu.VMEM((1, H), jnp.float32)],
    compiler_params=pltpu.CompilerParams(
        dimension_semantics=("arbitrary",),
    ),
)


def kernel(inputs, emb, W1, b1, W2, b2):
    embT = emb.T
    idx = inputs.astype(jnp.int32)
    embeds = _gather_call(idx, *([embT] * _GPER))
    return _mlp_call(embeds, W1, b1.reshape(1, H), W2.T,
                     b2.reshape(1, V))
